# baseline (speedup 1.0000x reference)
import jax
import jax.numpy as jnp
from jax.experimental import pallas as pl


_BATCH_BLOCK = 2048


def _dot_sigmoid_body(t_ref, c_ref, o_ref):
    o_ref[...] = jax.nn.sigmoid(jnp.sum(t_ref[...] * c_ref[...], axis=1))


def kernel(target_embeds, context_embeds):
    batch, dim = target_embeds.shape
    grid = (batch // _BATCH_BLOCK,)
    return pl.pallas_call(
        _dot_sigmoid_body,
        grid=grid,
        in_specs=[
            pl.BlockSpec((_BATCH_BLOCK, dim), lambda i: (i, 0)),
            pl.BlockSpec((_BATCH_BLOCK, dim), lambda i: (i, 0)),
        ],
        out_specs=pl.BlockSpec((_BATCH_BLOCK,), lambda i: (i,)),
        out_shape=jax.ShapeDtypeStruct((batch,), jnp.float32),
    )(target_embeds, context_embeds)
